# baseline (speedup 1.0000x reference)
import jax
import jax.numpy as jnp
from jax.experimental import pallas as pl
from jax.experimental.pallas import tpu as pltpu
from jax.experimental.pallas import tpu_sc as plsc

N = 100000
STATIC_DIM = 128
DYNAMIC_DIM = 128
OUT_DIM = STATIC_DIM + DYNAMIC_DIM
CHUNK = 1000
N_CHUNKS = N // CHUNK
N_WORKERS = 32


def _sc_concat(a_hbm, b_hbm, o_hbm, sems):
    w = jax.lax.axis_index("core") * 16 + jax.lax.axis_index("subcore")

    @pl.loop(w, N_CHUNKS, step=N_WORKERS)
    def _(c):
        base = pl.multiple_of(c * CHUNK, 8)
        rows = pl.ds(base, CHUNK)
        ca = pltpu.make_async_copy(
            a_hbm.at[rows, :], o_hbm.at[rows, pl.ds(0, STATIC_DIM)],
            sems.at[0])
        cb = pltpu.make_async_copy(
            b_hbm.at[rows, :], o_hbm.at[rows, pl.ds(STATIC_DIM, DYNAMIC_DIM)],
            sems.at[1])
        ca.start()
        cb.start()
        ca.wait()
        cb.wait()


def kernel(static_emb, dynamic_emb):
    mesh = plsc.VectorSubcoreMesh(core_axis_name="core",
                                  subcore_axis_name="subcore")
    sc_call = pl.kernel(
        _sc_concat,
        out_type=jax.ShapeDtypeStruct((N, OUT_DIM), jnp.float32),
        mesh=mesh,
        scratch_types=[pltpu.SemaphoreType.DMA((2,))],
    )
    return sc_call(static_emb, dynamic_emb)


# speedup vs baseline: 32.0680x; 32.0680x over previous
import jax
import jax.numpy as jnp
from jax.experimental import pallas as pl
from jax.experimental.pallas import tpu as pltpu
from jax.experimental.pallas import tpu_sc as plsc

N = 100000
STATIC_DIM = 128
DYNAMIC_DIM = 128
OUT_DIM = STATIC_DIM + DYNAMIC_DIM
CHUNK = 200
N_CHUNKS = N // CHUNK
N_WORKERS = 32


def _sc_concat(a_hbm, b_hbm, o_hbm, abuf, bbuf, sems):
    w = jax.lax.axis_index("core") * 16 + jax.lax.axis_index("subcore")

    @pl.loop(w, N_CHUNKS, step=N_WORKERS)
    def _(c):
        base = pl.multiple_of(c * CHUNK, 8)
        rows = pl.ds(base, CHUNK)
        ia = pltpu.make_async_copy(a_hbm.at[rows, :], abuf, sems.at[0])
        ib = pltpu.make_async_copy(b_hbm.at[rows, :], bbuf, sems.at[1])
        ia.start()
        ib.start()
        ia.wait()
        ib.wait()
        oa = pltpu.make_async_copy(
            abuf, o_hbm.at[rows, pl.ds(0, STATIC_DIM)], sems.at[0])
        ob = pltpu.make_async_copy(
            bbuf, o_hbm.at[rows, pl.ds(STATIC_DIM, DYNAMIC_DIM)], sems.at[1])
        oa.start()
        ob.start()
        oa.wait()
        ob.wait()


def kernel(static_emb, dynamic_emb):
    mesh = plsc.VectorSubcoreMesh(core_axis_name="core",
                                  subcore_axis_name="subcore")
    sc_call = pl.kernel(
        _sc_concat,
        out_type=jax.ShapeDtypeStruct((N, OUT_DIM), jnp.float32),
        mesh=mesh,
        scratch_types=[
            pltpu.VMEM((CHUNK, STATIC_DIM), jnp.float32),
            pltpu.VMEM((CHUNK, DYNAMIC_DIM), jnp.float32),
            pltpu.SemaphoreType.DMA((2,)),
        ],
    )
    return sc_call(static_emb, dynamic_emb)
